# Initial kernel scaffold; baseline (speedup 1.0000x reference)
import functools

import jax
import jax.numpy as jnp
from jax import lax
from jax.experimental import pallas as pl
from jax.experimental.pallas import tpu as pltpu
from jax.experimental.pallas import tpu_sc as plsc

N = 10000
NP = 10240
E = 160000
NC = 2
NS = 16
NW = NC * NS
EW = E // NW
RW = NP // NS
KE = 200
F1 = 256
F2 = 64
RB = 640

_mesh = plsc.VectorSubcoreMesh(
    core_axis_name="c", subcore_axis_name="s", num_cores=NC, num_subcores=NS
)


@functools.partial(
    pl.kernel,
    out_type=jax.ShapeDtypeStruct((NC, NP), jnp.float32),
    mesh=_mesh,
    scratch_types=[
        pltpu.VMEM((EW,), jnp.int32),
        pltpu.VMEM((EW + 8,), jnp.float32),
        pltpu.VMEM((RW,), jnp.float32),
        pltpu.VMEM_SHARED((NP,), jnp.float32),
    ],
)
def _sc_degree(dst_hbm, out_hbm, idx_v, ones_v, zeros_v, deg_sh):
    c = lax.axis_index("c")
    s = lax.axis_index("s")
    w = s * NC + c

    @pl.loop(0, EW + 8, step=16)
    def _(i):
        ones_v[pl.ds(i, 16)] = jnp.full((16,), 1.0, jnp.float32)

    @pl.loop(0, RW, step=16)
    def _(i):
        zeros_v[pl.ds(i, 16)] = jnp.zeros((16,), jnp.float32)

    pltpu.sync_copy(zeros_v, deg_sh.at[pl.ds(s * RW, RW)])
    plsc.subcore_barrier()
    pltpu.sync_copy(dst_hbm.at[pl.ds(w * EW, EW)], idx_v)
    pltpu.sync_copy(ones_v.at[pl.ds(0, EW)], deg_sh.at[idx_v], add=True)
    plsc.subcore_barrier()
    pltpu.sync_copy(deg_sh.at[pl.ds(s * RW, RW)], out_hbm.at[c, pl.ds(s * RW, RW)])


def _make_sc_aggregate(fc):
    @functools.partial(
        pl.kernel,
        out_type=jax.ShapeDtypeStruct((NC, NP, fc), jnp.float32),
        mesh=_mesh,
        scratch_types=[
            pltpu.VMEM((KE,), jnp.int32),
            pltpu.VMEM((KE,), jnp.int32),
            pltpu.VMEM((KE, fc), jnp.float32),
            pltpu.VMEM_SHARED((NP, fc), jnp.float32),
            pltpu.SemaphoreType.DMA,
        ],
    )
    def agg(y_hbm, src2_hbm, dst_hbm, out_hbm, si_v, di_v, rows_v, acc_sh, sem):
        c = lax.axis_index("c")
        s = lax.axis_index("s")
        w = s * NC + c

        pltpu.sync_copy(
            y_hbm.at[pl.ds(c * NP + s * RW, RW)], acc_sh.at[pl.ds(s * RW, RW)]
        )
        plsc.subcore_barrier()

        @pl.loop(0, EW // KE)
        def _(j):
            base = w * EW + j * KE
            pltpu.sync_copy(src2_hbm.at[c, pl.ds(base, KE)], si_v)
            pltpu.sync_copy(dst_hbm.at[pl.ds(base, KE)], di_v)
            pltpu.async_copy(y_hbm.at[si_v], rows_v, sem).wait()
            pltpu.sync_copy(rows_v, acc_sh.at[di_v], add=True)

        plsc.subcore_barrier()
        pltpu.sync_copy(
            acc_sh.at[pl.ds(s * RW, RW)], out_hbm.at[c, pl.ds(s * RW, RW), :]
        )

    return agg


_sc_agg1 = _make_sc_aggregate(F1 // 2)
_sc_agg2 = _make_sc_aggregate(F2 // 2)


def _dinv(deg_ref):
    return lax.rsqrt(deg_ref[0, :] + deg_ref[1, :] + 1.0)[:, None]


def _mm1_body(x_ref, w_ref, o_ref):
    o_ref[...] = jnp.dot(x_ref[...], w_ref[...], preferred_element_type=jnp.float32)


def _tc_matmul1(x, w1):
    return pl.pallas_call(
        _mm1_body,
        grid=(NP // RB,),
        in_specs=[
            pl.BlockSpec((RB, F1), lambda i: (i, 0)),
            pl.BlockSpec((F1, F1), lambda i: (0, 0)),
        ],
        out_specs=pl.BlockSpec((RB, F1), lambda i: (i, 0)),
        out_shape=jax.ShapeDtypeStruct((NP, F1), jnp.float32),
    )(x, w1)


def _scale_body(p_ref, deg_ref, o_ref):
    d = _dinv(deg_ref)
    o_ref[0] = p_ref[:, : F1 // 2] * d
    o_ref[1] = p_ref[:, F1 // 2 :] * d


def _tc_scale_split(p, deg2):
    return pl.pallas_call(
        _scale_body,
        grid=(NP // RB,),
        in_specs=[
            pl.BlockSpec((RB, F1), lambda i: (i, 0)),
            pl.BlockSpec((NC, RB), lambda i: (0, i)),
        ],
        out_specs=pl.BlockSpec((NC, RB, F1 // 2), lambda i: (0, i, 0)),
        out_shape=jax.ShapeDtypeStruct((NC, NP, F1 // 2), jnp.float32),
    )(p, deg2)


def _layer2_body(a_ref, deg_ref, b1_ref, w2_ref, o_ref):
    d = _dinv(deg_ref)
    h0 = jnp.maximum(a_ref[0] * d + b1_ref[0, : F1 // 2], 0.0)
    h1 = jnp.maximum(a_ref[1] * d + b1_ref[0, F1 // 2 :], 0.0)
    y = jnp.dot(h0, w2_ref[: F1 // 2, :], preferred_element_type=jnp.float32)
    y = y + jnp.dot(h1, w2_ref[F1 // 2 :, :], preferred_element_type=jnp.float32)
    y = y * d
    o_ref[0] = y[:, : F2 // 2]
    o_ref[1] = y[:, F2 // 2 :]


def _tc_layer2(agg1, deg2, b1, w2):
    return pl.pallas_call(
        _layer2_body,
        grid=(NP // RB,),
        in_specs=[
            pl.BlockSpec((NC, RB, F1 // 2), lambda i: (0, i, 0)),
            pl.BlockSpec((NC, RB), lambda i: (0, i)),
            pl.BlockSpec((1, F1), lambda i: (0, 0)),
            pl.BlockSpec((F1, F2), lambda i: (0, 0)),
        ],
        out_specs=pl.BlockSpec((NC, RB, F2 // 2), lambda i: (0, i, 0)),
        out_shape=jax.ShapeDtypeStruct((NC, NP, F2 // 2), jnp.float32),
    )(agg1, deg2, b1, w2)


def _final_body(a_ref, deg_ref, b2_ref, o_ref):
    d = _dinv(deg_ref)
    z = jnp.concatenate([a_ref[0], a_ref[1]], axis=1) * d + b2_ref[0, :]
    m = jnp.max(z, axis=1, keepdims=True)
    e = z - m
    lse = jnp.log(jnp.sum(jnp.exp(e), axis=1, keepdims=True))
    o_ref[...] = e - lse


def _tc_final(agg2, deg2, b2):
    return pl.pallas_call(
        _final_body,
        grid=(NP // RB,),
        in_specs=[
            pl.BlockSpec((NC, RB, F2 // 2), lambda i: (0, i, 0)),
            pl.BlockSpec((NC, RB), lambda i: (0, i)),
            pl.BlockSpec((1, F2), lambda i: (0, 0)),
        ],
        out_specs=pl.BlockSpec((RB, F2), lambda i: (i, 0)),
        out_shape=jax.ShapeDtypeStruct((NP, F2), jnp.float32),
    )(agg2, deg2, b2)


@jax.jit
def kernel(X, edge_index, W1, b1, W2, b2):
    src = edge_index[0]
    dst = edge_index[1]
    src2 = jnp.stack([src, src + NP])
    x_pad = jnp.pad(X, ((0, NP - N), (0, 0)))

    deg2 = _sc_degree(dst)
    p = _tc_matmul1(x_pad, W1)
    y1 = _tc_scale_split(p, deg2)
    agg1 = _sc_agg1(y1.reshape(NC * NP, F1 // 2), src2, dst)
    y2 = _tc_layer2(agg1, deg2, b1.reshape(1, F1), W2)
    agg2 = _sc_agg2(y2.reshape(NC * NP, F2 // 2), src2, dst)
    out = _tc_final(agg2, deg2, b2.reshape(1, F2))
    return out[:N]


# speedup vs baseline: 12.4394x; 12.4394x over previous
import functools

import jax
import jax.numpy as jnp
from jax import lax
from jax.experimental import pallas as pl
from jax.experimental.pallas import tpu as pltpu
from jax.experimental.pallas import tpu_sc as plsc

N = 10000
NP = 10240
E = 160000
NC = 2
NS = 16
NW = NC * NS
EW = E // NW
RW = NP // NS
KE = 200
F1 = 256
F2 = 64
RB = 640

_mesh = plsc.VectorSubcoreMesh(
    core_axis_name="c", subcore_axis_name="s", num_cores=NC, num_subcores=NS
)

_sc_params = pltpu.CompilerParams(use_tc_tiling_on_sc=False)


@functools.partial(
    pl.kernel,
    out_type=jax.ShapeDtypeStruct((NC * NP,), jnp.float32),
    mesh=_mesh,
    scratch_types=[
        pltpu.VMEM((EW,), jnp.int32),
        pltpu.VMEM((EW + 8,), jnp.float32),
        pltpu.VMEM((RW,), jnp.float32),
        pltpu.VMEM_SHARED((NP,), jnp.float32),
    ],
    compiler_params=_sc_params,
)
def _sc_degree(dst_hbm, out_hbm, idx_v, ones_v, zeros_v, deg_sh):
    c = lax.axis_index("c")
    s = lax.axis_index("s")
    w = s * NC + c

    @pl.loop(0, EW + 8, step=16)
    def _(i):
        ones_v[pl.ds(i, 16)] = jnp.full((16,), 1.0, jnp.float32)

    @pl.loop(0, RW, step=16)
    def _(i):
        zeros_v[pl.ds(i, 16)] = jnp.zeros((16,), jnp.float32)

    pltpu.sync_copy(zeros_v, deg_sh.at[pl.ds(s * RW, RW)])
    plsc.subcore_barrier()
    pltpu.sync_copy(dst_hbm.at[pl.ds(w * EW, EW)], idx_v)
    pltpu.sync_copy(ones_v.at[pl.ds(0, EW)], deg_sh.at[idx_v], add=True)
    plsc.subcore_barrier()
    pltpu.sync_copy(
        deg_sh.at[pl.ds(s * RW, RW)], out_hbm.at[pl.ds(c * NP + s * RW, RW)]
    )


def _make_sc_aggregate(fc):
    @functools.partial(
        pl.kernel,
        out_type=jax.ShapeDtypeStruct((NC * NP, fc), jnp.float32),
        mesh=_mesh,
        scratch_types=[
            pltpu.VMEM((KE,), jnp.int32),
            pltpu.VMEM((KE,), jnp.int32),
            pltpu.VMEM((KE, fc), jnp.float32),
            pltpu.VMEM_SHARED((NP, fc), jnp.float32),
            pltpu.SemaphoreType.DMA,
        ],
        compiler_params=_sc_params,
    )
    def agg(y_hbm, src2_hbm, dst_hbm, out_hbm, si_v, di_v, rows_v, acc_sh, sem):
        c = lax.axis_index("c")
        s = lax.axis_index("s")

        pltpu.sync_copy(
            y_hbm.at[pl.ds(c * NP + s * RW, RW)], acc_sh.at[pl.ds(s * RW, RW)]
        )
        plsc.subcore_barrier()

        ec = E // NS

        @pl.loop(0, ec // KE)
        def _(j):
            base = s * ec + j * KE
            pltpu.sync_copy(src2_hbm.at[pl.ds(c * E + base, KE)], si_v)
            pltpu.sync_copy(dst_hbm.at[pl.ds(base, KE)], di_v)
            pltpu.async_copy(y_hbm.at[si_v], rows_v, sem).wait()
            pltpu.sync_copy(rows_v, acc_sh.at[di_v], add=True)

        plsc.subcore_barrier()
        pltpu.sync_copy(
            acc_sh.at[pl.ds(s * RW, RW)], out_hbm.at[pl.ds(c * NP + s * RW, RW), :]
        )

    return agg


_sc_agg1 = _make_sc_aggregate(F1 // 2)
_sc_agg2 = _make_sc_aggregate(F2 // 2)


def _dinv(deg_ref):
    return lax.rsqrt(deg_ref[0, :] + deg_ref[1, :] + 1.0)[:, None]


def _dot(a, b):
    return jax.lax.dot(
        a, b, precision=jax.lax.Precision.HIGHEST,
        preferred_element_type=jnp.float32,
    )


def _mm1_body(x_ref, w_ref, o_ref):
    o_ref[...] = _dot(x_ref[...], w_ref[...])


def _tc_matmul1(x, w1):
    return pl.pallas_call(
        _mm1_body,
        grid=(NP // RB,),
        in_specs=[
            pl.BlockSpec((RB, F1), lambda i: (i, 0)),
            pl.BlockSpec((F1, F1), lambda i: (0, 0)),
        ],
        out_specs=pl.BlockSpec((RB, F1), lambda i: (i, 0)),
        out_shape=jax.ShapeDtypeStruct((NP, F1), jnp.float32),
    )(x, w1)


def _scale_body(p_ref, deg_ref, o_ref):
    d = _dinv(deg_ref)
    o_ref[0] = p_ref[:, : F1 // 2] * d
    o_ref[1] = p_ref[:, F1 // 2 :] * d


def _tc_scale_split(p, deg2):
    return pl.pallas_call(
        _scale_body,
        grid=(NP // RB,),
        in_specs=[
            pl.BlockSpec((RB, F1), lambda i: (i, 0)),
            pl.BlockSpec((NC, RB), lambda i: (0, i)),
        ],
        out_specs=pl.BlockSpec((NC, RB, F1 // 2), lambda i: (0, i, 0)),
        out_shape=jax.ShapeDtypeStruct((NC, NP, F1 // 2), jnp.float32),
    )(p, deg2)


def _layer2_body(a_ref, deg_ref, b1_ref, w2_ref, o_ref):
    d = _dinv(deg_ref)
    h0 = jnp.maximum(a_ref[0] * d + b1_ref[0, : F1 // 2], 0.0)
    h1 = jnp.maximum(a_ref[1] * d + b1_ref[0, F1 // 2 :], 0.0)
    y = _dot(h0, w2_ref[: F1 // 2, :]) + _dot(h1, w2_ref[F1 // 2 :, :])
    y = y * d
    o_ref[0] = y[:, : F2 // 2]
    o_ref[1] = y[:, F2 // 2 :]


def _tc_layer2(agg1, deg2, b1, w2):
    return pl.pallas_call(
        _layer2_body,
        grid=(NP // RB,),
        in_specs=[
            pl.BlockSpec((NC, RB, F1 // 2), lambda i: (0, i, 0)),
            pl.BlockSpec((NC, RB), lambda i: (0, i)),
            pl.BlockSpec((1, F1), lambda i: (0, 0)),
            pl.BlockSpec((F1, F2), lambda i: (0, 0)),
        ],
        out_specs=pl.BlockSpec((NC, RB, F2 // 2), lambda i: (0, i, 0)),
        out_shape=jax.ShapeDtypeStruct((NC, NP, F2 // 2), jnp.float32),
    )(agg1, deg2, b1, w2)


def _final_body(a_ref, deg_ref, b2_ref, o_ref):
    d = _dinv(deg_ref)
    z = jnp.concatenate([a_ref[0], a_ref[1]], axis=1) * d + b2_ref[0, :]
    m = jnp.max(z, axis=1, keepdims=True)
    e = z - m
    lse = jnp.log(jnp.sum(jnp.exp(e), axis=1, keepdims=True))
    o_ref[...] = e - lse


def _tc_final(agg2, deg2, b2):
    return pl.pallas_call(
        _final_body,
        grid=(NP // RB,),
        in_specs=[
            pl.BlockSpec((NC, RB, F2 // 2), lambda i: (0, i, 0)),
            pl.BlockSpec((NC, RB), lambda i: (0, i)),
            pl.BlockSpec((1, F2), lambda i: (0, 0)),
        ],
        out_specs=pl.BlockSpec((RB, F2), lambda i: (i, 0)),
        out_shape=jax.ShapeDtypeStruct((NP, F2), jnp.float32),
    )(agg2, deg2, b2)


@jax.jit
def kernel(X, edge_index, W1, b1, W2, b2):
    src = edge_index[0]
    dst = edge_index[1]
    src2 = jnp.concatenate([src, src + NP])
    x_pad = jnp.pad(X, ((0, NP - N), (0, 0)))

    deg2 = _sc_degree(dst).reshape(NC, NP)
    p = _tc_matmul1(x_pad, W1)
    y1 = _tc_scale_split(p, deg2)
    agg1 = _sc_agg1(y1.reshape(NC * NP, F1 // 2), src2, dst).reshape(NC, NP, F1 // 2)
    y2 = _tc_layer2(agg1, deg2, b1.reshape(1, F1), W2)
    agg2 = _sc_agg2(y2.reshape(NC * NP, F2 // 2), src2, dst).reshape(NC, NP, F2 // 2)
    out = _tc_final(agg2, deg2, b2.reshape(1, F2))
    return out[:N]
